# Initial kernel scaffold; baseline (speedup 1.0000x reference)
import jax
import jax.numpy as jnp
from jax.experimental import pallas as pl


def kernel(x, W1, b1, W2, b2, Wg, bg, Wc, bc):
    raise NotImplementedError("write your pallas kernel here")


# speedup vs baseline: 1.0338x; 1.0338x over previous
import functools

import jax
import jax.numpy as jnp
from jax.experimental import pallas as pl
from jax.experimental.pallas import tpu as pltpu

N = 4096
D = 2048
H = 2048
E = 8
C = 2048

TN = 256
TNS = 512
TC = 1024


def _router_body(x_ref, wgt_ref, bg_ref, s_ref):
    logits = jax.lax.dot_general(
        x_ref[...], wgt_ref[...], (((1,), (0,)), ((), ())),
        preferred_element_type=jnp.float32)
    logits = jnp.maximum(logits + bg_ref[...], 0.0)
    m = jnp.max(logits, axis=-1, keepdims=True)
    e = jnp.exp(logits - m)
    s_ref[...] = e / jnp.sum(e, axis=-1, keepdims=True)


def _fold_body(w2t_ref, wct_ref, b2_ref, bc_ref, mt_ref, v_ref):
    acc = jnp.dot(w2t_ref[0], wct_ref[...], preferred_element_type=jnp.float32)
    mt_ref[0] = acc.astype(jnp.bfloat16)
    vv = jnp.dot(b2_ref[0], wct_ref[...], preferred_element_type=jnp.float32)
    v_ref[0] = vv + bc_ref[...]


def _moe_body(x_ref, w1t_ref, b1_ref, mt_ref, v_ref, out_ref):
    h = jnp.dot(x_ref[...], w1t_ref[0], preferred_element_type=jnp.float32)
    h = jnp.maximum(h + b1_ref[0], 0.0).astype(jnp.bfloat16)
    acc = jnp.dot(h, mt_ref[0], preferred_element_type=jnp.float32)
    out_ref[...] = acc + v_ref[0]


@jax.jit
def kernel(x, W1, b1, W2, b2, Wg, bg, Wc, bc):
    x_bf = x.astype(jnp.bfloat16)
    w1t = jnp.transpose(W1, (0, 2, 1)).astype(jnp.bfloat16)
    w2t = jnp.transpose(W2, (0, 2, 1)).astype(jnp.bfloat16)
    wct = jnp.transpose(Wc).astype(jnp.bfloat16)
    wgt = jnp.transpose(Wg)
    b2_bf = b2.astype(jnp.bfloat16).reshape(E, 1, H)
    b1_3d = b1.reshape(E, 1, H)
    bg2 = bg.reshape(1, E)
    bc2 = bc.reshape(1, C)

    scores = pl.pallas_call(
        _router_body,
        grid=(N // TNS,),
        in_specs=[
            pl.BlockSpec((TNS, D), lambda i: (i, 0)),
            pl.BlockSpec((D, E), lambda i: (0, 0)),
            pl.BlockSpec((1, E), lambda i: (0, 0)),
        ],
        out_specs=pl.BlockSpec((TNS, E), lambda i: (i, 0)),
        out_shape=jax.ShapeDtypeStruct((N, E), jnp.float32),
    )(x, wgt, bg2)

    mt, v = pl.pallas_call(
        _fold_body,
        grid=(E, C // TC),
        in_specs=[
            pl.BlockSpec((1, H, H), lambda e, c: (e, 0, 0)),
            pl.BlockSpec((H, TC), lambda e, c: (0, c)),
            pl.BlockSpec((1, 1, H), lambda e, c: (e, 0, 0)),
            pl.BlockSpec((1, TC), lambda e, c: (0, c)),
        ],
        out_specs=[
            pl.BlockSpec((1, H, TC), lambda e, c: (e, 0, c)),
            pl.BlockSpec((1, 1, TC), lambda e, c: (e, 0, c)),
        ],
        out_shape=[
            jax.ShapeDtypeStruct((E, H, C), jnp.bfloat16),
            jax.ShapeDtypeStruct((E, 1, C), jnp.float32),
        ],
    )(w2t, wct, b2_bf, bc2)

    expert_outs = pl.pallas_call(
        _moe_body,
        grid=(E, N // TN),
        in_specs=[
            pl.BlockSpec((TN, D), lambda e, n: (n, 0)),
            pl.BlockSpec((1, D, H), lambda e, n: (e, 0, 0)),
            pl.BlockSpec((1, 1, H), lambda e, n: (e, 0, 0)),
            pl.BlockSpec((1, H, C), lambda e, n: (e, 0, 0)),
            pl.BlockSpec((1, 1, C), lambda e, n: (e, 0, 0)),
        ],
        out_specs=pl.BlockSpec((TN, C), lambda e, n: (n, e)),
        out_shape=jax.ShapeDtypeStruct((N, E * C), jnp.float32),
    )(x_bf, w1t, b1_3d, mt, v)

    return (expert_outs.reshape(N, E, C), scores)


# speedup vs baseline: 1.0969x; 1.0611x over previous
import jax
import jax.numpy as jnp
from jax.experimental import pallas as pl

N = 4096
D = 2048
H = 2048
E = 8
C = 2048

TN = 256
TNS = 512
TC = 1024

_NT = (((1,), (1,)), ((), ()))
_NN = (((1,), (0,)), ((), ()))


def _router_body(x_ref, wg_ref, bg_ref, s_ref):
    logits = jax.lax.dot_general(
        x_ref[...], wg_ref[...], _NT, preferred_element_type=jnp.float32)
    logits = jnp.maximum(logits + bg_ref[...], 0.0)
    m = jnp.max(logits, axis=-1, keepdims=True)
    e = jnp.exp(logits - m)
    s_ref[...] = e / jnp.sum(e, axis=-1, keepdims=True)


def _fold_body(wc_ref, w2_ref, b2_ref, bc_ref, m_ref, v_ref):
    m_ref[0] = jax.lax.dot_general(
        wc_ref[...], w2_ref[0], _NN,
        preferred_element_type=jnp.float32).astype(jnp.bfloat16)
    vv = jax.lax.dot_general(
        b2_ref[0], wc_ref[...], _NT, preferred_element_type=jnp.float32)
    v_ref[0] = vv + bc_ref[...]


def _moe_body(x_ref, w1_ref, b1_ref, m_ref, v_ref, out_ref):
    h = jax.lax.dot_general(
        x_ref[...], w1_ref[0], _NT, preferred_element_type=jnp.float32)
    h = jnp.maximum(h + b1_ref[0], 0.0).astype(jnp.bfloat16)
    acc = jax.lax.dot_general(
        h, m_ref[0], _NT, preferred_element_type=jnp.float32)
    out_ref[...] = acc + v_ref[0]


@jax.jit
def kernel(x, W1, b1, W2, b2, Wg, bg, Wc, bc):
    x_bf = x.astype(jnp.bfloat16)
    w1_bf = W1.astype(jnp.bfloat16)
    w2_bf = W2.astype(jnp.bfloat16)
    wc_bf = Wc.astype(jnp.bfloat16)
    b2_bf = b2.astype(jnp.bfloat16).reshape(E, 1, H)
    b1_3d = b1.reshape(E, 1, H)
    bg2 = bg.reshape(1, E)
    bc2 = bc.reshape(1, C)

    scores = pl.pallas_call(
        _router_body,
        grid=(N // TNS,),
        in_specs=[
            pl.BlockSpec((TNS, D), lambda i: (i, 0)),
            pl.BlockSpec((E, D), lambda i: (0, 0)),
            pl.BlockSpec((1, E), lambda i: (0, 0)),
        ],
        out_specs=pl.BlockSpec((TNS, E), lambda i: (i, 0)),
        out_shape=jax.ShapeDtypeStruct((N, E), jnp.float32),
    )(x, Wg, bg2)

    m, v = pl.pallas_call(
        _fold_body,
        grid=(E, C // TC),
        in_specs=[
            pl.BlockSpec((TC, H), lambda e, c: (c, 0)),
            pl.BlockSpec((1, H, H), lambda e, c: (e, 0, 0)),
            pl.BlockSpec((1, 1, H), lambda e, c: (e, 0, 0)),
            pl.BlockSpec((1, TC), lambda e, c: (0, c)),
        ],
        out_specs=[
            pl.BlockSpec((1, TC, H), lambda e, c: (e, c, 0)),
            pl.BlockSpec((1, 1, TC), lambda e, c: (e, 0, c)),
        ],
        out_shape=[
            jax.ShapeDtypeStruct((E, C, H), jnp.bfloat16),
            jax.ShapeDtypeStruct((E, 1, C), jnp.float32),
        ],
    )(wc_bf, w2_bf, b2_bf, bc2)

    expert_outs = pl.pallas_call(
        _moe_body,
        grid=(E, N // TN),
        in_specs=[
            pl.BlockSpec((TN, D), lambda e, n: (n, 0)),
            pl.BlockSpec((1, H, D), lambda e, n: (e, 0, 0)),
            pl.BlockSpec((1, 1, H), lambda e, n: (e, 0, 0)),
            pl.BlockSpec((1, C, H), lambda e, n: (e, 0, 0)),
            pl.BlockSpec((1, 1, C), lambda e, n: (e, 0, 0)),
        ],
        out_specs=pl.BlockSpec((TN, C), lambda e, n: (n, e)),
        out_shape=jax.ShapeDtypeStruct((N, E * C), jnp.float32),
    )(x_bf, w1_bf, b1_3d, m, v)

    return (expert_outs.reshape(N, E, C), scores)


# speedup vs baseline: 1.1007x; 1.0035x over previous
import jax
import jax.numpy as jnp
from jax.experimental import pallas as pl

N = 4096
D = 2048
H = 2048
E = 8
C = 2048

TN = 256
TNS = 512
TC = 1024

_NT = (((1,), (1,)), ((), ()))
_NN = (((1,), (0,)), ((), ()))


def _router_body(x_ref, wg_ref, bg_ref, s_ref, xbf_ref):
    logits = jax.lax.dot_general(
        x_ref[...], wg_ref[...], _NT, preferred_element_type=jnp.float32)
    logits = jnp.maximum(logits + bg_ref[...], 0.0)
    m = jnp.max(logits, axis=-1, keepdims=True)
    e = jnp.exp(logits - m)
    s_ref[...] = e / jnp.sum(e, axis=-1, keepdims=True)
    xbf_ref[...] = x_ref[...].astype(jnp.bfloat16)


def _cast_body(w_ref, wbf_ref):
    wbf_ref[0] = w_ref[0].astype(jnp.bfloat16)


def _fold_body(wc_ref, w2_ref, b2_ref, bc_ref, m_ref, v_ref):
    m_ref[0] = jax.lax.dot_general(
        wc_ref[...], w2_ref[0], _NN,
        preferred_element_type=jnp.float32).astype(jnp.bfloat16)
    vv = jax.lax.dot_general(
        b2_ref[0], wc_ref[...], _NT, preferred_element_type=jnp.float32)
    v_ref[0] = vv + bc_ref[...]


def _moe_body(x_ref, w1_ref, b1_ref, m_ref, v_ref, out_ref):
    h = jax.lax.dot_general(
        x_ref[...], w1_ref[0], _NT, preferred_element_type=jnp.float32)
    h = jnp.maximum(h + b1_ref[0], 0.0).astype(jnp.bfloat16)
    acc = jax.lax.dot_general(
        h, m_ref[0], _NT, preferred_element_type=jnp.float32)
    out_ref[...] = acc + v_ref[0]


@jax.jit
def kernel(x, W1, b1, W2, b2, Wg, bg, Wc, bc):
    wc_bf = Wc.astype(jnp.bfloat16)
    b2_bf = b2.astype(jnp.bfloat16).reshape(E, 1, H)
    b1_3d = b1.reshape(E, 1, H)
    bg2 = bg.reshape(1, E)
    bc2 = bc.reshape(1, C)

    scores, x_bf = pl.pallas_call(
        _router_body,
        grid=(N // TNS,),
        in_specs=[
            pl.BlockSpec((TNS, D), lambda i: (i, 0)),
            pl.BlockSpec((E, D), lambda i: (0, 0)),
            pl.BlockSpec((1, E), lambda i: (0, 0)),
        ],
        out_specs=[
            pl.BlockSpec((TNS, E), lambda i: (i, 0)),
            pl.BlockSpec((TNS, D), lambda i: (i, 0)),
        ],
        out_shape=[
            jax.ShapeDtypeStruct((N, E), jnp.float32),
            jax.ShapeDtypeStruct((N, D), jnp.bfloat16),
        ],
    )(x, Wg, bg2)

    def _cast3(w):
        e, r, c = w.shape
        return pl.pallas_call(
            _cast_body,
            grid=(e,),
            in_specs=[pl.BlockSpec((1, r, c), lambda i: (i, 0, 0))],
            out_specs=pl.BlockSpec((1, r, c), lambda i: (i, 0, 0)),
            out_shape=jax.ShapeDtypeStruct((e, r, c), jnp.bfloat16),
        )(w)

    w1_bf = _cast3(W1)
    w2_bf = _cast3(W2)

    m, v = pl.pallas_call(
        _fold_body,
        grid=(E, C // TC),
        in_specs=[
            pl.BlockSpec((TC, H), lambda e, c: (c, 0)),
            pl.BlockSpec((1, H, H), lambda e, c: (e, 0, 0)),
            pl.BlockSpec((1, 1, H), lambda e, c: (e, 0, 0)),
            pl.BlockSpec((1, TC), lambda e, c: (0, c)),
        ],
        out_specs=[
            pl.BlockSpec((1, TC, H), lambda e, c: (e, c, 0)),
            pl.BlockSpec((1, 1, TC), lambda e, c: (e, 0, c)),
        ],
        out_shape=[
            jax.ShapeDtypeStruct((E, C, H), jnp.bfloat16),
            jax.ShapeDtypeStruct((E, 1, C), jnp.float32),
        ],
    )(wc_bf, w2_bf, b2_bf, bc2)

    expert_outs = pl.pallas_call(
        _moe_body,
        grid=(E, N // TN),
        in_specs=[
            pl.BlockSpec((TN, D), lambda e, n: (n, 0)),
            pl.BlockSpec((1, H, D), lambda e, n: (e, 0, 0)),
            pl.BlockSpec((1, 1, H), lambda e, n: (e, 0, 0)),
            pl.BlockSpec((1, C, H), lambda e, n: (e, 0, 0)),
            pl.BlockSpec((1, 1, C), lambda e, n: (e, 0, 0)),
        ],
        out_specs=pl.BlockSpec((TN, C), lambda e, n: (n, e)),
        out_shape=jax.ShapeDtypeStruct((N, E * C), jnp.float32),
    )(x_bf, w1_bf, b1_3d, m, v)

    return (expert_outs.reshape(N, E, C), scores)


# speedup vs baseline: 1.1850x; 1.0766x over previous
import jax
import jax.numpy as jnp
from jax.experimental import pallas as pl

N = 4096
D = 2048
H = 2048
E = 8
C = 2048

TN = 512
TNS = 256

_NT = (((1,), (1,)), ((), ()))
_NN = (((1,), (0,)), ((), ()))


def _router_body(x_ref, wg_ref, bg_ref, w1_ref, s_ref, xbf_ref, w1bf_ref):
    logits = jax.lax.dot_general(
        x_ref[...], wg_ref[...], _NT, preferred_element_type=jnp.float32)
    logits = jnp.maximum(logits + bg_ref[...], 0.0)
    m = jnp.max(logits, axis=-1, keepdims=True)
    e = jnp.exp(logits - m)
    s_ref[...] = e / jnp.sum(e, axis=-1, keepdims=True)
    xbf_ref[...] = x_ref[...].astype(jnp.bfloat16)
    w1bf_ref[0] = w1_ref[0].astype(jnp.bfloat16)


def _fold_body(wc_ref, w2_ref, b2_ref, bc_ref, m_ref, v_ref):
    k = pl.program_id(1)
    w2b = w2_ref[0].astype(jnp.bfloat16)
    vv = jax.lax.dot_general(
        b2_ref[0], wc_ref[...], _NT, preferred_element_type=jnp.float32)
    ch = C // 2
    for j in range(2):
        part = jax.lax.dot_general(
            wc_ref[j * ch:(j + 1) * ch, :], w2b, _NN,
            preferred_element_type=jnp.float32)

        @pl.when(k == 0)
        def _(part=part, j=j):
            m_ref[0, j * ch:(j + 1) * ch, :] = part.astype(jnp.bfloat16)

        @pl.when(k != 0)
        def _(part=part, j=j):
            old = m_ref[0, j * ch:(j + 1) * ch, :].astype(jnp.float32)
            m_ref[0, j * ch:(j + 1) * ch, :] = (old + part).astype(jnp.bfloat16)

    @pl.when(k == 0)
    def _():
        v_ref[0] = vv + bc_ref[...]

    @pl.when(k != 0)
    def _():
        v_ref[0] = v_ref[0] + vv


def _moe_body(x_ref, w1_ref, b1_ref, m_ref, v_ref, out_ref):
    xb = x_ref[...]
    hh = H // 2
    acc = None
    for k in range(2):
        hk = jax.lax.dot_general(
            xb, w1_ref[0, k * hh:(k + 1) * hh, :], _NT,
            preferred_element_type=jnp.float32)
        hk = jnp.maximum(hk + b1_ref[0, :, k * hh:(k + 1) * hh],
                         0.0).astype(jnp.bfloat16)
        pk = jax.lax.dot_general(
            hk, m_ref[0, :, k * hh:(k + 1) * hh], _NT,
            preferred_element_type=jnp.float32)
        acc = pk if acc is None else acc + pk
    out_ref[...] = acc + v_ref[0]


@jax.jit
def kernel(x, W1, b1, W2, b2, Wg, bg, Wc, bc):
    wc_bf = Wc.astype(jnp.bfloat16)
    b2_bf = b2.astype(jnp.bfloat16).reshape(E, 1, H)
    b1_3d = b1.reshape(E, 1, H)
    bg2 = bg.reshape(1, E)
    bc2 = bc.reshape(1, C)

    nsteps = N // TNS
    wh = (H * E) // nsteps
    scores, x_bf, w1_bf = pl.pallas_call(
        _router_body,
        grid=(nsteps,),
        in_specs=[
            pl.BlockSpec((TNS, D), lambda i: (i, 0)),
            pl.BlockSpec((E, D), lambda i: (0, 0)),
            pl.BlockSpec((1, E), lambda i: (0, 0)),
            pl.BlockSpec((1, wh, D), lambda i: (i, 0, 0)),
        ],
        out_specs=[
            pl.BlockSpec((TNS, E), lambda i: (i, 0)),
            pl.BlockSpec((TNS, D), lambda i: (i, 0)),
            pl.BlockSpec((1, wh, D), lambda i: (i, 0, 0)),
        ],
        out_shape=[
            jax.ShapeDtypeStruct((N, E), jnp.float32),
            jax.ShapeDtypeStruct((N, D), jnp.bfloat16),
            jax.ShapeDtypeStruct((nsteps, wh, D), jnp.bfloat16),
        ],
    )(x, Wg, bg2, W1.reshape(nsteps, wh, D))
    w1_bf = w1_bf.reshape(E, H, D)

    oh = H // 2
    m, v = pl.pallas_call(
        _fold_body,
        grid=(E, 2),
        in_specs=[
            pl.BlockSpec((C, oh), lambda e, k: (0, k)),
            pl.BlockSpec((1, oh, H), lambda e, k: (e, k, 0)),
            pl.BlockSpec((1, 1, oh), lambda e, k: (e, 0, k)),
            pl.BlockSpec((1, C), lambda e, k: (0, 0)),
        ],
        out_specs=[
            pl.BlockSpec((1, C, H), lambda e, k: (e, 0, 0)),
            pl.BlockSpec((1, 1, C), lambda e, k: (e, 0, 0)),
        ],
        out_shape=[
            jax.ShapeDtypeStruct((E, C, H), jnp.bfloat16),
            jax.ShapeDtypeStruct((E, 1, C), jnp.float32),
        ],
    )(wc_bf, W2, b2_bf, bc2)

    expert_outs = pl.pallas_call(
        _moe_body,
        grid=(E, N // TN),
        in_specs=[
            pl.BlockSpec((TN, D), lambda e, n: (n, 0)),
            pl.BlockSpec((1, H, D), lambda e, n: (e, 0, 0)),
            pl.BlockSpec((1, 1, H), lambda e, n: (e, 0, 0)),
            pl.BlockSpec((1, C, H), lambda e, n: (e, 0, 0)),
            pl.BlockSpec((1, 1, C), lambda e, n: (e, 0, 0)),
        ],
        out_specs=pl.BlockSpec((TN, C), lambda e, n: (n, e)),
        out_shape=jax.ShapeDtypeStruct((N, E * C), jnp.float32),
    )(x_bf, w1_bf, b1_3d, m, v)

    return (expert_outs.reshape(N, E, C), scores)
